# Initial kernel scaffold; baseline (speedup 1.0000x reference)
import jax
import jax.numpy as jnp
from jax.experimental import pallas as pl


def kernel(x, gate_w, e_score_correction_bias, w_gate_up, w_down, ws_gate_up, ws_down):
    raise NotImplementedError("write your pallas kernel here")


# speedup vs baseline: 1.0846x; 1.0846x over previous
import functools

import jax
import jax.numpy as jnp
from jax.experimental import pallas as pl
from jax.experimental.pallas import tpu as pltpu

EP = 128
SCALE = 2.5


def _sigmoid(x):
    return 1.0 / (1.0 + jnp.exp(-x))


def _routing_body(x_ref, gwt_ref, bias_ref, cw_ref, *, E, TT):
    logits = jax.lax.dot_general(
        x_ref[...].astype(jnp.bfloat16), gwt_ref[...].astype(jnp.bfloat16),
        (((1,), (0,)), ((), ())),
        preferred_element_type=jnp.float32)
    lane = jax.lax.broadcasted_iota(jnp.int32, (TT, EP), 1)
    valid = lane < E
    scores = _sigmoid(logits)
    biased = jnp.where(valid, scores + bias_ref[...], -jnp.inf)
    m1 = jnp.max(biased, axis=1, keepdims=True)
    i1 = jnp.min(jnp.where(biased == m1, lane, EP), axis=1, keepdims=True)
    b2 = jnp.where(lane == i1, -jnp.inf, biased)
    m2 = jnp.max(b2, axis=1, keepdims=True)
    i2 = jnp.min(jnp.where(b2 == m2, lane, EP), axis=1, keepdims=True)
    w1 = jnp.sum(jnp.where(lane == i1, scores, 0.0), axis=1, keepdims=True)
    w2 = jnp.sum(jnp.where(lane == i2, scores, 0.0), axis=1, keepdims=True)
    denom = w1 + w2 + 1e-20
    w1 = w1 / denom * SCALE
    w2 = w2 / denom * SCALE
    cw = jnp.where(lane == i1, w1, 0.0) + jnp.where(lane == i2, w2, 0.0)
    cw_ref[...] = jnp.where(lane == E, 1.0, cw)


def _moe_body(xb_ref, cw_ref, wgu_ref, wdn_ref, out_ref, *, FF, TT):
    e = pl.program_id(0)
    t = pl.program_id(1)

    @pl.when(jnp.logical_and(e == 0, t == 0))
    def _():
        out_ref[...] = jnp.zeros_like(out_ref)

    xs = xb_ref[pl.ds(t * TT, TT), :]
    gu = jax.lax.dot_general(
        xs, wgu_ref[0], (((1,), (0,)), ((), ())),
        preferred_element_type=jnp.float32)
    g = gu[:, :FF]
    u = gu[:, FF:]
    act = (g * _sigmoid(g) * u).astype(jnp.bfloat16)
    eo = jax.lax.dot_general(
        act, wdn_ref[0], (((1,), (0,)), ((), ())),
        preferred_element_type=jnp.float32)
    lane = jax.lax.broadcasted_iota(jnp.int32, (TT, EP), 1)
    w = jnp.sum(jnp.where(lane == e, cw_ref[pl.ds(t * TT, TT), :], 0.0),
                axis=1, keepdims=True)
    out_ref[pl.ds(t * TT, TT), :] += eo * w


def kernel(x, gate_w, e_score_correction_bias, w_gate_up, w_down,
           ws_gate_up, ws_down):
    T, H = x.shape
    E = gate_w.shape[0]
    FF = w_down.shape[1]
    NE = E + 1
    TT = 256 if T % 256 == 0 else T
    NT = T // TT

    gwt = jnp.zeros((H, EP), jnp.float32).at[:, :E].set(gate_w.T)
    bias = jnp.zeros((1, EP), jnp.float32).at[0, :E].set(e_score_correction_bias)
    xb = x.astype(jnp.bfloat16)
    wgu_all = jnp.concatenate(
        [w_gate_up, ws_gate_up[None]], axis=0).astype(jnp.bfloat16)
    wdn_all = jnp.concatenate(
        [w_down, ws_down[None]], axis=0).astype(jnp.bfloat16)

    cw = pl.pallas_call(
        functools.partial(_routing_body, E=E, TT=TT),
        grid=(NT,),
        in_specs=[
            pl.BlockSpec((TT, H), lambda t: (t, 0)),
            pl.BlockSpec((H, EP), lambda t: (0, 0)),
            pl.BlockSpec((1, EP), lambda t: (0, 0)),
        ],
        out_specs=pl.BlockSpec((TT, EP), lambda t: (t, 0)),
        out_shape=jax.ShapeDtypeStruct((T, EP), jnp.float32),
    )(x, gwt, bias)

    out = pl.pallas_call(
        functools.partial(_moe_body, FF=FF, TT=TT),
        grid=(NE, NT),
        in_specs=[
            pl.BlockSpec((T, H), lambda e, t: (0, 0)),
            pl.BlockSpec((T, EP), lambda e, t: (0, 0)),
            pl.BlockSpec((1, H, 2 * FF), lambda e, t: (e, 0, 0)),
            pl.BlockSpec((1, FF, H), lambda e, t: (e, 0, 0)),
        ],
        out_specs=pl.BlockSpec((T, H), lambda e, t: (0, 0)),
        out_shape=jax.ShapeDtypeStruct((T, H), jnp.float32),
        compiler_params=pltpu.CompilerParams(
            dimension_semantics=("arbitrary", "arbitrary")),
    )(xb, cw, wgu_all, wdn_all)
    return out
